# Initial kernel scaffold; baseline (speedup 1.0000x reference)
import jax
import jax.numpy as jnp
from jax.experimental import pallas as pl


def kernel(x, edge_index1, edge_index2, W_self1, W_neigh1, b1, W_self2, W_neigh2, b2, W_fin, b_fin):
    raise NotImplementedError("write your pallas kernel here")


# speedup vs baseline: 4.2010x; 4.2010x over previous
import jax
import jax.numpy as jnp
from jax import lax
from jax.experimental import pallas as pl
from jax.experimental.pallas import tpu as pltpu
from jax.experimental.pallas import tpu_sc as plsc

N = 10000
NP = 10240
D = 128
H = 128
OUT = 2
E = 320000

NC = 2
NS = 16
NW = NC * NS
CH = 128
NCHUNK = 79
EPAD = NW * NCHUNK * CH
ROWS_PT = NP // NS
DEGW = 16

BLK = 1024


def _sc_agg_body(x_hbm, src_hbm, dst_hbm, zf_hbm, out_hbm,
                 src_v, dst_v, rows_v, acc, sem):
    c = lax.axis_index("c")
    s = lax.axis_index("s")

    pltpu.sync_copy(src_hbm.at[c, s], src_v)
    pltpu.sync_copy(dst_hbm.at[c, s], dst_v)

    r0 = s * ROWS_PT
    pltpu.sync_copy(zf_hbm.at[pl.ds(r0, ROWS_PT)], acc.at[pl.ds(r0, ROWS_PT)])
    plsc.subcore_barrier()

    def chunk(j, carry):
        pltpu.async_copy(x_hbm.at[src_v.at[j]], rows_v, sem).wait()
        pltpu.sync_copy(rows_v, acc.at[dst_v.at[j]], add=True)
        return carry

    lax.fori_loop(0, NCHUNK, chunk, 0)
    plsc.subcore_barrier()

    pltpu.sync_copy(acc.at[pl.ds(r0, ROWS_PT)], out_hbm.at[c, pl.ds(r0, ROWS_PT)])


def _sc_agg(x, src, dst, zf):
    mesh = plsc.VectorSubcoreMesh(
        core_axis_name="c", subcore_axis_name="s",
        num_cores=NC, num_subcores=NS)
    return pl.kernel(
        _sc_agg_body,
        out_type=jax.ShapeDtypeStruct((NC, NP, D), jnp.float32),
        mesh=mesh,
        scratch_types=[
            pltpu.VMEM((NCHUNK, CH), jnp.int32),
            pltpu.VMEM((NCHUNK, CH), jnp.int32),
            pltpu.VMEM((CH, D), jnp.float32),
            pltpu.VMEM_SHARED((NP, D), jnp.float32),
            pltpu.SemaphoreType.DMA,
        ])(x, src, dst, zf)


def _sc_deg_body(dst1_hbm, dst2_hbm, zd_hbm, ones_hbm,
                 deg1_hbm, deg2_hbm,
                 dst1_v, dst2_v, ones_v, dacc1, dacc2):
    c = lax.axis_index("c")
    s = lax.axis_index("s")

    pltpu.sync_copy(dst1_hbm.at[c, s], dst1_v)
    pltpu.sync_copy(dst2_hbm.at[c, s], dst2_v)
    pltpu.sync_copy(ones_hbm, ones_v)

    r0 = s * ROWS_PT
    pltpu.sync_copy(zd_hbm.at[pl.ds(r0, ROWS_PT)], dacc1.at[pl.ds(r0, ROWS_PT)])
    pltpu.sync_copy(zd_hbm.at[pl.ds(r0, ROWS_PT)], dacc2.at[pl.ds(r0, ROWS_PT)])
    plsc.subcore_barrier()

    def chunk(j, carry):
        pltpu.sync_copy(ones_v, dacc1.at[dst1_v.at[j]], add=True)
        pltpu.sync_copy(ones_v, dacc2.at[dst2_v.at[j]], add=True)
        return carry

    lax.fori_loop(0, NCHUNK, chunk, 0)
    plsc.subcore_barrier()

    pltpu.sync_copy(dacc1.at[pl.ds(r0, ROWS_PT)], deg1_hbm.at[c, pl.ds(r0, ROWS_PT)])
    pltpu.sync_copy(dacc2.at[pl.ds(r0, ROWS_PT)], deg2_hbm.at[c, pl.ds(r0, ROWS_PT)])


def _sc_deg(dst1, dst2, zd, ones):
    mesh = plsc.VectorSubcoreMesh(
        core_axis_name="c", subcore_axis_name="s",
        num_cores=NC, num_subcores=NS)
    return pl.kernel(
        _sc_deg_body,
        out_type=[jax.ShapeDtypeStruct((NC, NP, DEGW), jnp.float32),
                  jax.ShapeDtypeStruct((NC, NP, DEGW), jnp.float32)],
        mesh=mesh,
        compiler_params=pltpu.CompilerParams(use_tc_tiling_on_sc=False),
        scratch_types=[
            pltpu.VMEM((NCHUNK, CH), jnp.int32),
            pltpu.VMEM((NCHUNK, CH), jnp.int32),
            pltpu.VMEM((CH, DEGW), jnp.float32),
            pltpu.VMEM_SHARED((NP, DEGW), jnp.float32),
            pltpu.VMEM_SHARED((NP, DEGW), jnp.float32),
        ])(dst1, dst2, zd, ones)


def _tc_layer_body(x_ref, p0_ref, p1_ref, d0_ref, d1_ref,
                   ws_ref, wn_ref, b_ref, o_ref):
    cnt = jnp.sum(d0_ref[...] + d1_ref[...], axis=1, keepdims=True) * (1.0 / DEGW)
    deg = jnp.maximum(cnt, 1.0)
    agg = (p0_ref[...] + p1_ref[...]) / deg
    h = (jnp.dot(x_ref[...], ws_ref[...], preferred_element_type=jnp.float32)
         + jnp.dot(agg, wn_ref[...], preferred_element_type=jnp.float32)
         + b_ref[...])
    o_ref[...] = jnp.maximum(h, 0.0)


def _tc_final_body(x_ref, p0_ref, p1_ref, d0_ref, d1_ref,
                   ws_ref, wn_ref, b_ref, wf_ref, bf_ref, o_ref):
    cnt = jnp.sum(d0_ref[...] + d1_ref[...], axis=1, keepdims=True) * (1.0 / DEGW)
    deg = jnp.maximum(cnt, 1.0)
    agg = (p0_ref[...] + p1_ref[...]) / deg
    h = (jnp.dot(x_ref[...], ws_ref[...], preferred_element_type=jnp.float32)
         + jnp.dot(agg, wn_ref[...], preferred_element_type=jnp.float32)
         + b_ref[...])
    h = jnp.maximum(h, 0.0)
    o_ref[...] = (jnp.dot(h, wf_ref[...], preferred_element_type=jnp.float32)
                  + bf_ref[...])


def _row_blocked(width):
    return pl.BlockSpec((BLK, width), lambda i: (i, 0))


def _full(shape):
    return pl.BlockSpec(shape, lambda i: tuple(0 for _ in shape))


def _tc_layer(x, p0, p1, d0, d1, ws, wn, b):
    return pl.pallas_call(
        _tc_layer_body,
        grid=(NP // BLK,),
        in_specs=[_row_blocked(D), _row_blocked(D), _row_blocked(D),
                  _row_blocked(DEGW), _row_blocked(DEGW),
                  _full((D, H)), _full((D, H)), _full((1, H))],
        out_specs=_row_blocked(H),
        out_shape=jax.ShapeDtypeStruct((NP, H), jnp.float32),
    )(x, p0, p1, d0, d1, ws, wn, b)


def _tc_final(x, p0, p1, d0, d1, ws, wn, b, wf, bf):
    return pl.pallas_call(
        _tc_final_body,
        grid=(NP // BLK,),
        in_specs=[_row_blocked(H), _row_blocked(H), _row_blocked(H),
                  _row_blocked(DEGW), _row_blocked(DEGW),
                  _full((H, H)), _full((H, H)), _full((1, H)),
                  _full((H, H)), _full((1, H))],
        out_specs=_row_blocked(H),
        out_shape=jax.ShapeDtypeStruct((NP, H), jnp.float32),
    )(x, p0, p1, d0, d1, ws, wn, b, wf, bf)


def _pad_edges(e):
    pad = jnp.full((EPAD - E,), NP - 1, jnp.int32)
    return jnp.concatenate([e, pad]).reshape(NC, NS, NCHUNK, CH)


def kernel(x, edge_index1, edge_index2, W_self1, W_neigh1, b1,
           W_self2, W_neigh2, b2, W_fin, b_fin):
    src1 = _pad_edges(edge_index1[0])
    dst1 = _pad_edges(edge_index1[1])
    src2 = _pad_edges(edge_index2[0])
    dst2 = _pad_edges(edge_index2[1])

    zf = jnp.zeros((NP, D), jnp.float32)
    zd = jnp.zeros((NP, DEGW), jnp.float32)
    ones = jnp.ones((CH, DEGW), jnp.float32)
    x_p = jnp.pad(x, ((0, NP - N), (0, 0)))

    agg1 = _sc_agg(x_p, src1, dst1, zf)
    deg1, deg2 = _sc_deg(dst1, dst2, zd, ones)

    h1 = _tc_layer(x_p, agg1[0], agg1[1], deg1[0], deg1[1],
                   W_self1, W_neigh1, b1.reshape(1, H))

    agg2 = _sc_agg(h1, src2, dst2, zf)

    wf = jnp.zeros((H, H), jnp.float32).at[:, :OUT].set(W_fin)
    bf = jnp.zeros((1, H), jnp.float32).at[0, :OUT].set(b_fin)
    out = _tc_final(h1, agg2[0], agg2[1], deg2[0], deg2[1],
                    W_self2, W_neigh2, b2.reshape(1, H), wf, bf)
    return out[:N, :OUT]
